# baseline (speedup 1.0000x reference)
import functools

import jax
import jax.numpy as jnp
from jax import lax
from jax.experimental import pallas as pl
from jax.experimental.pallas import tpu as pltpu
from jax.experimental.pallas import tpu_sc as plsc

B, N, D = 8, 576, 64
N_TOK = B * N
K = 8192
DP = 2 * D

TT = 512
NT = N_TOK // TT
CB = 512
NCH = K // CB

NC, NS, LANES = 2, 16, 16
NW = NC * NS
BPW = N_TOK // NW
CH = BPW // 2


def _argmin_body(z_ref, emb_ref, idx_ref, embp_ref, en_ref):
    @pl.when(pl.program_id(0) == 0)
    def _():
        e = emb_ref[...]
        en_ref[...] = jnp.sum(e * e, axis=1)[None, :]

    z = z_ref[...]
    zn = jnp.sum(z * z, axis=1, keepdims=True)
    z2x = z + z

    def chunk(k, carry):
        run_v, run_i = carry
        e = emb_ref[pl.ds(k * CB, CB), :]
        en = en_ref[:, pl.ds(k * CB, CB)]
        dot2 = lax.dot_general(z2x, e, (((1,), (1,)), ((), ())),
                               preferred_element_type=jnp.float32)
        dch = (zn + en) - dot2
        better = dch < run_v
        run_v = jnp.minimum(run_v, dch)
        run_i = jnp.where(better, jnp.float32(1.0) * k, run_i)
        return run_v, run_i

    init = (jnp.full((TT, CB), jnp.inf, jnp.float32),
            jnp.zeros((TT, CB), jnp.float32))
    run_v, run_i = lax.fori_loop(0, NCH, chunk, init)

    rmin = jnp.min(run_v, axis=1, keepdims=True)
    lane = lax.broadcasted_iota(jnp.int32, (TT, CB), 1).astype(jnp.float32)
    colv = run_i * jnp.float32(CB) + lane
    cand = jnp.where(run_v == rmin, colv, jnp.float32(2 * K))
    idx_ref[...] = jnp.min(cand, axis=1).astype(jnp.int32)

    @pl.when(pl.program_id(0) == NT - 1)
    def _():
        embp_ref[:, :D] = emb_ref[...]
        embp_ref[:, D:] = jnp.zeros((K, D), jnp.float32)


_tc_argmin = pl.pallas_call(
    _argmin_body,
    grid=(NT,),
    in_specs=[
        pl.BlockSpec((TT, D), lambda i: (i, 0)),
        pl.BlockSpec((K, D), lambda i: (0, 0)),
    ],
    out_specs=[
        pl.BlockSpec((TT,), lambda i: (i,)),
        pl.BlockSpec((K, DP), lambda i: (0, 0)),
    ],
    out_shape=[
        jax.ShapeDtypeStruct((N_TOK,), jnp.int32),
        jax.ShapeDtypeStruct((K, DP), jnp.float32),
    ],
    scratch_shapes=[pltpu.VMEM((1, K), jnp.float32)],
)


_sc_mesh = plsc.VectorSubcoreMesh(core_axis_name="c", subcore_axis_name="s")


@functools.partial(
    pl.kernel,
    mesh=_sc_mesh,
    out_type=(
        jax.ShapeDtypeStruct((N_TOK, D), jnp.float32),
        jax.ShapeDtypeStruct((NW * LANES,), jnp.float32),
    ),
    scratch_types=[
        pltpu.VMEM((CH,), jnp.int32),
        pltpu.VMEM((CH,), jnp.int32),
        pltpu.VMEM((CH, DP), jnp.float32),
        pltpu.VMEM((CH, DP), jnp.float32),
        pltpu.VMEM((BPW, D), jnp.float32),
        pltpu.VMEM((BPW, D), jnp.float32),
        pltpu.VMEM((LANES,), jnp.float32),
        pltpu.SemaphoreType.DMA,
    ],
)
def _sc_gather_loss(emb_hbm, idx_hbm, z_hbm, out_hbm, psum_hbm,
                    idx_a, idx_b, rows_a, rows_b, z_v, out_v, acc_v, sem):
    wid = lax.axis_index("s") * NC + lax.axis_index("c")
    base = wid * BPW
    pltpu.sync_copy(idx_hbm.at[pl.ds(base, CH)], idx_a)
    pltpu.sync_copy(idx_hbm.at[pl.ds(base + CH, CH)], idx_b)
    ca = pltpu.async_copy(emb_hbm.at[idx_a], rows_a, sem)
    cb = pltpu.async_copy(emb_hbm.at[idx_b], rows_b, sem)
    pltpu.sync_copy(z_hbm.at[pl.ds(base, BPW)], z_v)
    ca.wait()
    cb.wait()

    def make_body(rows, roff):
        def body(r, acc):
            for c in range(D // LANES):
                q = rows[r - roff, pl.ds(c * LANES, LANES)]
                t = z_v[r, pl.ds(c * LANES, LANES)]
                out_v[r, pl.ds(c * LANES, LANES)] = q
                dd = q - t
                acc = acc + dd * dd
            return acc
        return body

    acc = lax.fori_loop(0, CH, make_body(rows_a, 0),
                        jnp.zeros((LANES,), jnp.float32))
    acc = lax.fori_loop(CH, BPW, make_body(rows_b, CH), acc)
    acc_v[...] = acc

    pltpu.sync_copy(out_v, out_hbm.at[pl.ds(base, BPW)])
    pltpu.sync_copy(acc_v, psum_hbm.at[pl.ds(wid * LANES, LANES)])


def kernel(z, emb_weight):
    z = z.astype(jnp.float32)
    zf = z.reshape(-1, D)
    idx_flat, emb_p = _tc_argmin(zf, emb_weight)
    quant_flat, psums = _sc_gather_loss(emb_p, idx_flat, zf)
    quantized = quant_flat.reshape(z.shape)
    mse = jnp.sum(psums) / float(N_TOK * D)
    zero = jnp.array(0.0, dtype=jnp.float32)
    loss = 0.25 * mse + 1.0 * mse + 0.0 * zero
    q_indices = idx_flat.reshape(B, N)
    return (z, emb_weight, quantized, q_indices, loss, mse, mse,
            zero, zero, zero)


# speedup vs baseline: 1.5040x; 1.5040x over previous
import functools

import jax
import jax.numpy as jnp
from jax import lax
from jax.experimental import pallas as pl
from jax.experimental.pallas import tpu as pltpu
from jax.experimental.pallas import tpu_sc as plsc

B, N, D = 8, 576, 64
N_TOK = B * N
K = 8192
DP = 2 * D

TT = 512
NT = N_TOK // TT
CB = 512
NCH = K // CB

NC, NS, LANES = 2, 16, 16
NW = NC * NS
BPW = N_TOK // NW
CH = BPW // 2


def _argmin_body(z_ref, emb_ref, idx_ref, embp_ref, en_ref):
    @pl.when(pl.program_id(0) == 0)
    def _():
        e0 = emb_ref[...]
        en_ref[...] = jnp.sum(e0 * e0, axis=1)[None, :]

    z = z_ref[...]
    e = emb_ref[...]
    zn = jnp.sum(z * z, axis=1, keepdims=True)
    dot2 = lax.dot_general(z + z, e, (((1,), (1,)), ((), ())),
                           preferred_element_type=jnp.float32)
    d = (zn + en_ref[...]) - dot2
    rmin = jnp.min(d, axis=1, keepdims=True)
    ONE = jnp.int32(0x3F800000)
    col = lax.broadcasted_iota(jnp.int32, d.shape, 1)
    pseudo = lax.bitcast_convert_type(ONE + col, jnp.float32)
    cand = jnp.where(d == rmin, pseudo, jnp.float32(4.0))
    m = jnp.min(cand, axis=1)
    idx_ref[...] = lax.bitcast_convert_type(m, jnp.int32) - ONE

    @pl.when(pl.program_id(0) == NT - 1)
    def _():
        embp_ref[:, :D] = emb_ref[...]
        embp_ref[:, D:] = jnp.zeros((K, D), jnp.float32)


_tc_argmin = pl.pallas_call(
    _argmin_body,
    grid=(NT,),
    in_specs=[
        pl.BlockSpec((TT, D), lambda i: (i, 0)),
        pl.BlockSpec((K, D), lambda i: (0, 0)),
    ],
    out_specs=[
        pl.BlockSpec((TT,), lambda i: (i,)),
        pl.BlockSpec((K, DP), lambda i: (0, 0)),
    ],
    out_shape=[
        jax.ShapeDtypeStruct((N_TOK,), jnp.int32),
        jax.ShapeDtypeStruct((K, DP), jnp.float32),
    ],
    scratch_shapes=[pltpu.VMEM((1, K), jnp.float32)],
)


_sc_mesh = plsc.VectorSubcoreMesh(core_axis_name="c", subcore_axis_name="s")


@functools.partial(
    pl.kernel,
    mesh=_sc_mesh,
    out_type=(
        jax.ShapeDtypeStruct((N_TOK, D), jnp.float32),
        jax.ShapeDtypeStruct((NW * LANES,), jnp.float32),
    ),
    scratch_types=[
        pltpu.VMEM((CH,), jnp.int32),
        pltpu.VMEM((CH,), jnp.int32),
        pltpu.VMEM((CH, DP), jnp.float32),
        pltpu.VMEM((CH, DP), jnp.float32),
        pltpu.VMEM((BPW, D), jnp.float32),
        pltpu.VMEM((BPW, D), jnp.float32),
        pltpu.VMEM((LANES,), jnp.float32),
        pltpu.SemaphoreType.DMA,
    ],
)
def _sc_gather_loss(emb_hbm, idx_hbm, z_hbm, out_hbm, psum_hbm,
                    idx_a, idx_b, rows_a, rows_b, z_v, out_v, acc_v, sem):
    wid = lax.axis_index("s") * NC + lax.axis_index("c")
    base = wid * BPW
    pltpu.sync_copy(idx_hbm.at[pl.ds(base, CH)], idx_a)
    pltpu.sync_copy(idx_hbm.at[pl.ds(base + CH, CH)], idx_b)
    ca = pltpu.async_copy(emb_hbm.at[idx_a], rows_a, sem)
    cb = pltpu.async_copy(emb_hbm.at[idx_b], rows_b, sem)
    pltpu.sync_copy(z_hbm.at[pl.ds(base, BPW)], z_v)
    ca.wait()
    cb.wait()

    def make_body(rows, roff):
        def body(r, acc):
            for c in range(D // LANES):
                q = rows[r - roff, pl.ds(c * LANES, LANES)]
                t = z_v[r, pl.ds(c * LANES, LANES)]
                out_v[r, pl.ds(c * LANES, LANES)] = q
                dd = q - t
                acc = acc + dd * dd
            return acc
        return body

    acc = lax.fori_loop(0, CH, make_body(rows_a, 0),
                        jnp.zeros((LANES,), jnp.float32))
    acc = lax.fori_loop(CH, BPW, make_body(rows_b, CH), acc)
    acc_v[...] = acc

    pltpu.sync_copy(out_v, out_hbm.at[pl.ds(base, BPW)])
    pltpu.sync_copy(acc_v, psum_hbm.at[pl.ds(wid * LANES, LANES)])


def kernel(z, emb_weight):
    z = z.astype(jnp.float32)
    zf = z.reshape(-1, D)
    idx_flat, emb_p = _tc_argmin(zf, emb_weight)
    quant_flat, psums = _sc_gather_loss(emb_p, idx_flat, zf)
    quantized = quant_flat.reshape(z.shape)
    mse = jnp.sum(psums) / float(N_TOK * D)
    zero = jnp.array(0.0, dtype=jnp.float32)
    loss = 0.25 * mse + 1.0 * mse + 0.0 * zero
    q_indices = idx_flat.reshape(B, N)
    return (z, emb_weight, quantized, q_indices, loss, mse, mse,
            zero, zero, zero)
